# baseline (speedup 1.0000x reference)
import functools

import jax
import jax.numpy as jnp
from jax import lax
from jax.experimental import pallas as pl
from jax.experimental.pallas import tpu as pltpu
from jax.experimental.pallas import tpu_sc as plsc
from jax.experimental.layout import Format, Layout, with_layout_constraint

_DIM = 768
_INTER = 256
_E = 64
_T = 2048
_C = 1.0

_GATE_TILE = 256
_N_GATE = _T // _GATE_TILE
_NP = 2 * _T
_ROW_TILE = 128
_N_TILES = _NP // _ROW_TILE
_G_ITEMS = _N_TILES + _E
_NW = 32
_PW = _NP // _NW
_TW = _T // _NW


def _gate_body(x_ref, wg_ref, bg_ref, ltri_ref, eye_ref, l64x_ref, l64i_ref,
               route_ref, off_ref, cum_ref, base_ref):
    i = pl.program_id(0)

    @pl.when(i == 0)
    def _():
        base_ref[...] = jnp.zeros_like(base_ref)

    x = x_ref[...]
    logits = lax.dot_general(x, wg_ref[...], (((1,), (1,)), ((), ())),
                             preferred_element_type=jnp.float32)
    m = jnp.max(logits, axis=1, keepdims=True)
    ex = jnp.exp(logits - m)
    p = ex / jnp.sum(ex, axis=1, keepdims=True)
    b = p + bg_ref[...]

    lane = lax.broadcasted_iota(jnp.int32, b.shape, 1)
    m1 = jnp.max(b, axis=1, keepdims=True)
    i1 = jnp.min(jnp.where(b == m1, lane, _E), axis=1, keepdims=True)
    oh1 = lane == i1
    b2 = jnp.where(oh1, jnp.float32(-1e30), b)
    m2 = jnp.max(b2, axis=1, keepdims=True)
    i2 = jnp.min(jnp.where(b2 == m2, lane, _E), axis=1, keepdims=True)
    oh2 = lane == i2
    w1 = jnp.sum(jnp.where(oh1, p, 0.0), axis=1, keepdims=True)
    w2 = jnp.sum(jnp.where(oh2, p, 0.0), axis=1, keepdims=True)

    mm = oh1.astype(jnp.float32) + oh2.astype(jnp.float32)
    csum = lax.dot_general(ltri_ref[...], mm, (((1,), (0,)), ((), ())),
                           preferred_element_type=jnp.float32)
    baserow = base_ref[0:1, :]
    cb = csum + baserow
    r1 = jnp.sum(jnp.where(oh1, cb, 0.0), axis=1, keepdims=True)
    r2 = jnp.sum(jnp.where(oh2, cb, 0.0), axis=1, keepdims=True)

    v = jnp.concatenate(
        [r1, r2, i1.astype(jnp.float32), i2.astype(jnp.float32), w1, w2],
        axis=1)
    vt = lax.dot_general(v, eye_ref[...], (((0,), (0,)), ((), ())),
                         preferred_element_type=jnp.float32)
    route_ref[0] = vt

    newbase = baserow + jnp.sum(mm, axis=0, keepdims=True)
    base_ref[0:1, :] = newbase
    off_ref[...] = lax.dot_general(newbase, l64x_ref[...],
                                   (((1,), (1,)), ((), ())),
                                   preferred_element_type=jnp.float32)
    cum_ref[...] = lax.dot_general(newbase, l64i_ref[...],
                                   (((1,), (1,)), ((), ())),
                                   preferred_element_type=jnp.float32)


def _expert_ffn(x, w1, b1, w3, b3, w2, b2, c_exp):
    xc = x.astype(w1.dtype)
    s1 = lax.dot_general(xc, w1, (((1,), (1,)), ((), ())),
                         preferred_element_type=jnp.float32) + b1
    s1 = s1 * lax.logistic(s1)
    s3 = lax.dot_general(xc, w3, (((1,), (1,)), ((), ())),
                         preferred_element_type=jnp.float32) + b3
    xs = s1 * s3
    xt = jnp.sqrt(jnp.clip(
        jnp.sum(xs * xs, axis=1, keepdims=True) + c_exp, 1e-8))
    h = jnp.concatenate([xt, xs], axis=1).astype(w2.dtype)
    os_ = lax.dot_general(h, w2, (((1,), (1,)), ((), ())),
                          preferred_element_type=jnp.float32) + b2
    ot = jnp.sqrt(jnp.clip(
        jnp.sum(os_ * os_, axis=1, keepdims=True) + _C, 1e-8))
    return jnp.concatenate([ot, os_], axis=1)


def _grouped_body(tiles_ref, experts_ref, starts_ref, ends_ref, inits_ref,
                  xs_ref, w1_ref, b1_ref, w3_ref, b3_ref, w2_ref, b2_ref,
                  ys_ref):
    g = pl.program_id(0)
    ye = _expert_ffn(xs_ref[...], w1_ref[:, 0, 0, :], b1_ref[0],
                     w3_ref[:, 0, 0, :], b3_ref[0], w2_ref[:, 0, 0, :],
                     b2_ref[0], _C)
    pos = _ROW_TILE * tiles_ref[g] + lax.broadcasted_iota(
        jnp.int32, (_ROW_TILE, 1), 0)
    mask = (pos >= starts_ref[g]) & (pos < ends_ref[g])

    @pl.when(inits_ref[g] == 1)
    def _():
        ys_ref[...] = jnp.where(mask, ye, 0.0)

    @pl.when(inits_ref[g] == 0)
    def _():
        ys_ref[...] = jnp.where(mask, ye, ys_ref[...])


def _dispatch_body(e_hbm, r_hbm, off_hbm, cum_hbm, x_hbm,
                   xs_hbm, dest_hbm, tiles_hbm, experts_hbm, starts_hbm,
                   ends_hbm, inits_hbm,
                   ev, rv, offs, cum, destv, xv,
                   first_v, nbase_v, pa, pb, t96, e96, s96, en96, in96, sem):
    wid = lax.axis_index("s") * 2 + lax.axis_index("c")
    base_j = wid * _PW
    tok0 = (wid % (_NW // 2)) * _PW

    pltpu.sync_copy(e_hbm.at[pl.ds(base_j, _PW)], ev)
    pltpu.sync_copy(r_hbm.at[pl.ds(base_j, _PW)], rv)
    pltpu.sync_copy(off_hbm, offs)
    pltpu.sync_copy(cum_hbm, cum)

    for c in range(_PW // 16):
        ej = ev[pl.ds(c * 16, 16)]
        oj = plsc.load_gather(offs, [ej])
        destv[pl.ds(c * 16, 16)] = oj + rv[pl.ds(c * 16, 16)]
    pltpu.sync_copy(destv, dest_hbm.at[pl.ds(base_j, _PW)])

    pltpu.sync_copy(x_hbm.at[pl.ds(tok0, _PW)], xv)
    pltpu.async_copy(xv, xs_hbm.at[destv], sem).wait()

    @pl.when(wid == 0)
    def _():
        for c in range(_N_TILES // 16):
            pvec = (lax.iota(jnp.int32, 16) + 16 * c) * _ROW_TILE
            fcnt = jnp.zeros((16,), jnp.int32)
            lcnt = jnp.zeros((16,), jnp.int32)
            for c2 in range(_E // 16):
                cchunk = cum[pl.ds(c2 * 16, 16)]
                for k in range(16):
                    ce = cchunk[k]
                    fcnt = fcnt + (ce <= pvec).astype(jnp.int32)
                    lcnt = lcnt + (
                        ce <= pvec + (_ROW_TILE - 1)).astype(jnp.int32)
            first_v[pl.ds(c * 16, 16)] = fcnt
            nbase_v[pl.ds(c * 16, 16)] = lcnt - fcnt + 1
        zeros16 = jnp.zeros((16,), jnp.int32)
        pa[pl.ds(0, 16)] = zeros16
        for c in range(_N_TILES // 16):
            pa[pl.ds(16 + c * 16, 16)] = nbase_v[pl.ds(c * 16, 16)]
        cur, nxt = pa, pb
        for kk in (1, 2, 4, 8, 16):
            nxt[pl.ds(0, 16)] = zeros16
            for c in range(_N_TILES // 16):
                nxt[pl.ds(16 + c * 16, 16)] = (
                    cur[pl.ds(16 + c * 16, 16)]
                    + cur[pl.ds(16 + c * 16 - kk, 16)])
            cur, nxt = nxt, cur
        lastchunk = cur[pl.ds(16 + _N_TILES - 16, 16)]
        tot_items = lastchunk[15]
        for c in range(_N_TILES // 16):
            inc = cur[pl.ds(16 + c * 16, 16)]
            ni = nbase_v[pl.ds(c * 16, 16)]
            nbase_v[pl.ds(c * 16, 16)] = inc - ni
        for c in range(_G_ITEMS // 16):
            gvec = lax.iota(jnp.int32, 16) + 16 * c
            tcnt = jnp.zeros((16,), jnp.int32)
            for c2 in range(_N_TILES // 16):
                bchunk = nbase_v[pl.ds(c2 * 16, 16)]
                for k in range(16):
                    tcnt = tcnt + (bchunk[k] <= gvec).astype(jnp.int32)
            tg = tcnt - 1
            bg_ = plsc.load_gather(nbase_v, [tg])
            fg = plsc.load_gather(first_v, [tg])
            eg = fg + gvec - bg_
            eg = jnp.minimum(jnp.maximum(eg, 0), _E - 1)
            og = plsc.load_gather(offs, [eg])
            cg = plsc.load_gather(cum, [eg])
            st = jnp.maximum(og, tg * _ROW_TILE)
            en = jnp.minimum(cg, tg * _ROW_TILE + _ROW_TILE)
            valid = gvec < tot_items
            st = jnp.where(valid, st, 0)
            en = jnp.where(valid, en, 0)
            init = (gvec == bg_).astype(jnp.int32)
            t96[pl.ds(c * 16, 16)] = tg
            e96[pl.ds(c * 16, 16)] = eg
            s96[pl.ds(c * 16, 16)] = st
            en96[pl.ds(c * 16, 16)] = en
            in96[pl.ds(c * 16, 16)] = init
        pltpu.sync_copy(t96, tiles_hbm)
        pltpu.sync_copy(e96, experts_hbm)
        pltpu.sync_copy(s96, starts_hbm)
        pltpu.sync_copy(en96, ends_hbm)
        pltpu.sync_copy(in96, inits_hbm)


def _sc_dispatch(e_flat, r_flat, offs_arr, cum_arr, x):
    f32, i32 = jnp.float32, jnp.int32
    mesh = plsc.VectorSubcoreMesh(core_axis_name="c", subcore_axis_name="s")
    call = pl.kernel(
        _dispatch_body,
        out_type=(
            jax.ShapeDtypeStruct((_NP, _DIM), f32),
            jax.ShapeDtypeStruct((_NP,), i32),
            jax.ShapeDtypeStruct((_G_ITEMS,), i32),
            jax.ShapeDtypeStruct((_G_ITEMS,), i32),
            jax.ShapeDtypeStruct((_G_ITEMS,), i32),
            jax.ShapeDtypeStruct((_G_ITEMS,), i32),
            jax.ShapeDtypeStruct((_G_ITEMS,), i32),
        ),
        mesh=mesh,
        scratch_types=[
            pltpu.VMEM((_PW,), i32),
            pltpu.VMEM((_PW,), i32),
            pltpu.VMEM((_E,), i32),
            pltpu.VMEM((_E,), i32),
            pltpu.VMEM((_PW,), i32),
            pltpu.VMEM((_PW, _DIM), f32),
            pltpu.VMEM((_N_TILES,), i32),
            pltpu.VMEM((_N_TILES,), i32),
            pltpu.VMEM((16 + _N_TILES,), i32),
            pltpu.VMEM((16 + _N_TILES,), i32),
            pltpu.VMEM((_G_ITEMS,), i32),
            pltpu.VMEM((_G_ITEMS,), i32),
            pltpu.VMEM((_G_ITEMS,), i32),
            pltpu.VMEM((_G_ITEMS,), i32),
            pltpu.VMEM((_G_ITEMS,), i32),
            pltpu.SemaphoreType.DMA,
        ],
        compiler_params=pltpu.CompilerParams(needs_layout_passes=False),
    )
    return call(e_flat, r_flat, offs_arr, cum_arr, x)


def _combine_body(ys_hbm, dest_hbm, w_hbm, acc_hbm,
                  d0, d1, wv0, wv1, rows0, rows1, accv, sem0, sem1):
    wid = lax.axis_index("s") * 2 + lax.axis_index("c")
    half = _TW // 2
    for bat in range(2):
        tbase = wid * _TW + bat * half
        pltpu.sync_copy(dest_hbm.at[pl.ds(tbase, half)], d0)
        pltpu.sync_copy(dest_hbm.at[pl.ds(_T + tbase, half)], d1)
        pltpu.sync_copy(w_hbm.at[pl.ds(tbase, half)], wv0)
        pltpu.sync_copy(w_hbm.at[pl.ds(_T + tbase, half)], wv1)
        c0 = pltpu.async_copy(ys_hbm.at[d0], rows0, sem0)
        c1 = pltpu.async_copy(ys_hbm.at[d1], rows1, sem1)
        c0.wait()
        c1.wait()

        def body(t, _):
            tsplat = jnp.full((16,), t, jnp.int32)
            w0 = plsc.load_gather(wv0, [tsplat])
            w1 = plsc.load_gather(wv1, [tsplat])
            for c in range(_DIM // 16):
                sl = pl.ds(c * 16, 16)
                accv[t, sl] = rows0[t, sl] * w0 + rows1[t, sl] * w1
            return 0

        lax.fori_loop(0, half, body, 0)
        pltpu.sync_copy(accv, acc_hbm.at[pl.ds(tbase, half)])


def _sc_combine(ys, dest, w_flat):
    f32, i32 = jnp.float32, jnp.int32
    half = _TW // 2
    mesh = plsc.VectorSubcoreMesh(core_axis_name="c", subcore_axis_name="s")
    call = pl.kernel(
        _combine_body,
        out_type=jax.ShapeDtypeStruct((_T, _DIM), f32),
        mesh=mesh,
        scratch_types=[
            pltpu.VMEM((half,), i32),
            pltpu.VMEM((half,), i32),
            pltpu.VMEM((half,), f32),
            pltpu.VMEM((half,), f32),
            pltpu.VMEM((half, _DIM), f32),
            pltpu.VMEM((half, _DIM), f32),
            pltpu.VMEM((half, _DIM), f32),
            pltpu.SemaphoreType.DMA,
            pltpu.SemaphoreType.DMA,
        ],
        compiler_params=pltpu.CompilerParams(needs_layout_passes=False),
    )
    return call(ys, dest, w_flat)


def _shared_body(x_ref, ws1_ref, bs1_ref, ws3_ref, bs3_ref, ws2_ref, bs2_ref,
                 out_ref):
    out_ref[...] = _expert_ffn(x_ref[...], ws1_ref[...], bs1_ref[...],
                               ws3_ref[...], bs3_ref[...], ws2_ref[...],
                               bs2_ref[...], _C)


def _midpoint_norm(z):
    lane = lax.broadcasted_iota(jnp.int32, z.shape, 1)
    sgn = jnp.where(lane == 0, -1.0, 1.0)
    inner = jnp.sum(z * z * sgn, axis=1, keepdims=True)
    return z / jnp.sqrt(jnp.clip(-inner, 1e-8))


def _final_body(shared_ref, acc_ref, out_ref):
    routed = _midpoint_norm(acc_ref[...])
    out_ref[...] = _midpoint_norm(shared_ref[...] + routed)


def kernel(x, Wg, bg, W1, b1, W3, b3, W2, b2, Ws1, bs1, Ws3, bs3, Ws2, bs2):
    f32, i32 = jnp.float32, jnp.int32
    wg_ext = jnp.concatenate([jnp.zeros((_E, 1), f32), Wg], axis=1)
    ltri = jnp.tri(_GATE_TILE, k=-1, dtype=f32)
    eye = jnp.eye(_GATE_TILE, dtype=f32)
    l64x = jnp.tri(_E, k=-1, dtype=f32)
    l64i = jnp.tri(_E, k=0, dtype=f32)

    route, offv, cumv = pl.pallas_call(
        _gate_body,
        grid=(_N_GATE,),
        in_specs=[
            pl.BlockSpec((_GATE_TILE, _DIM), lambda i: (i, 0)),
            pl.BlockSpec((_E, _DIM), lambda i: (0, 0)),
            pl.BlockSpec((1, _E), lambda i: (0, 0)),
            pl.BlockSpec((_GATE_TILE, _GATE_TILE), lambda i: (0, 0)),
            pl.BlockSpec((_GATE_TILE, _GATE_TILE), lambda i: (0, 0)),
            pl.BlockSpec((_E, _E), lambda i: (0, 0)),
            pl.BlockSpec((_E, _E), lambda i: (0, 0)),
        ],
        out_specs=[
            pl.BlockSpec((1, 6, _GATE_TILE), lambda i: (i, 0, 0)),
            pl.BlockSpec((1, _E), lambda i: (0, 0)),
            pl.BlockSpec((1, _E), lambda i: (0, 0)),
        ],
        out_shape=[
            jax.ShapeDtypeStruct((_N_GATE, 6, _GATE_TILE), f32),
            jax.ShapeDtypeStruct((1, _E), f32),
            jax.ShapeDtypeStruct((1, _E), f32),
        ],
        scratch_shapes=[pltpu.VMEM((8, _E), f32)],
    )(x, wg_ext, bg.reshape(1, _E), ltri, eye, l64x, l64i)

    r_flat = jnp.concatenate(
        [route[:, 0, :].reshape(-1), route[:, 1, :].reshape(-1)]).astype(i32)
    e_flat = jnp.concatenate(
        [route[:, 2, :].reshape(-1), route[:, 3, :].reshape(-1)]).astype(i32)
    w_flat = jnp.concatenate(
        [route[:, 4, :].reshape(-1), route[:, 5, :].reshape(-1)])
    offs_arr = offv.reshape(_E).astype(i32)
    cum_arr = cumv.reshape(_E).astype(i32)

    xs, dest, tiles, experts, starts, ends, inits = _sc_dispatch(
        e_flat, r_flat, offs_arr, cum_arr, x)

    def _native(w):
        wt = w.transpose(1, 0, 2)
        wt = with_layout_constraint(wt, Layout(major_to_minor=(0, 1, 2)))
        return wt.reshape(wt.shape[0], _E, 1, wt.shape[2])

    w1n = _native(W1)
    w3n = _native(W3)
    w2n = _native(W2)

    grid_spec = pltpu.PrefetchScalarGridSpec(
        num_scalar_prefetch=5,
        grid=(_G_ITEMS,),
        in_specs=[
            pl.BlockSpec((_ROW_TILE, _DIM),
                         lambda g, tr, er, sr, enr, ir: (tr[g], 0)),
            pl.BlockSpec((_INTER - 1, 1, 1, _DIM),
                         lambda g, tr, er, sr, enr, ir: (0, er[g], 0, 0)),
            pl.BlockSpec((1, 1, _INTER - 1),
                         lambda g, tr, er, sr, enr, ir: (er[g], 0, 0)),
            pl.BlockSpec((_INTER - 1, 1, 1, _DIM),
                         lambda g, tr, er, sr, enr, ir: (0, er[g], 0, 0)),
            pl.BlockSpec((1, 1, _INTER - 1),
                         lambda g, tr, er, sr, enr, ir: (er[g], 0, 0)),
            pl.BlockSpec((_DIM - 1, 1, 1, _INTER),
                         lambda g, tr, er, sr, enr, ir: (0, er[g], 0, 0)),
            pl.BlockSpec((1, 1, _DIM - 1),
                         lambda g, tr, er, sr, enr, ir: (er[g], 0, 0)),
        ],
        out_specs=pl.BlockSpec((_ROW_TILE, _DIM),
                               lambda g, tr, er, sr, enr, ir: (tr[g], 0)),
    )
    ys = pl.pallas_call(
        _grouped_body,
        grid_spec=grid_spec,
        out_shape=jax.ShapeDtypeStruct((_NP, _DIM), f32),
    )(tiles, experts, starts, ends, inits, xs,
      w1n, b1.reshape(_E, 1, _INTER - 1), w3n, b3.reshape(_E, 1, _INTER - 1),
      w2n, b2.reshape(_E, 1, _DIM - 1))

    acc = _sc_combine(ys, dest, w_flat)

    shared = pl.pallas_call(
        _shared_body,
        grid=(_T // 128,),
        in_specs=[
            pl.BlockSpec((128, _DIM), lambda i: (i, 0)),
            pl.BlockSpec((_INTER - 1, _DIM), lambda i: (0, 0)),
            pl.BlockSpec((1, _INTER - 1), lambda i: (0, 0)),
            pl.BlockSpec((_INTER - 1, _DIM), lambda i: (0, 0)),
            pl.BlockSpec((1, _INTER - 1), lambda i: (0, 0)),
            pl.BlockSpec((_DIM - 1, _INTER), lambda i: (0, 0)),
            pl.BlockSpec((1, _DIM - 1), lambda i: (0, 0)),
        ],
        out_specs=pl.BlockSpec((128, _DIM), lambda i: (i, 0)),
        out_shape=jax.ShapeDtypeStruct((_T, _DIM), f32),
    )(x, Ws1, bs1.reshape(1, _INTER - 1), Ws3, bs3.reshape(1, _INTER - 1),
      Ws2, bs2.reshape(1, _DIM - 1))

    out = pl.pallas_call(
        _final_body,
        grid=(_T // 128,),
        in_specs=[
            pl.BlockSpec((128, _DIM), lambda i: (i, 0)),
            pl.BlockSpec((128, _DIM), lambda i: (i, 0)),
        ],
        out_specs=pl.BlockSpec((128, _DIM), lambda i: (i, 0)),
        out_shape=jax.ShapeDtypeStruct((_T, _DIM), f32),
    )(shared, acc)
    return out


# speedup vs baseline: 1.8037x; 1.8037x over previous
import functools

import jax
import jax.numpy as jnp
from jax import lax
from jax.experimental import pallas as pl
from jax.experimental.pallas import tpu as pltpu
from jax.experimental.pallas import tpu_sc as plsc
from jax.experimental.layout import Format, Layout, with_layout_constraint

_DIM = 768
_INTER = 256
_E = 64
_T = 2048
_C = 1.0

_GATE_TILE = 256
_N_GATE = _T // _GATE_TILE
_NP = 2 * _T
_ROW_TILE = 128
_N_TILES = _NP // _ROW_TILE
_G_ITEMS = _N_TILES + _E
_NW = 32
_PW = _NP // _NW
_TW = _T // _NW


def _gate_body(x_ref, wg_ref, bg_ref, ltri_ref, eye_ref, l64x_ref, l64i_ref,
               route_ref, off_ref, cum_ref, base_ref):
    i = pl.program_id(0)

    @pl.when(i == 0)
    def _():
        base_ref[...] = jnp.zeros_like(base_ref)

    x = x_ref[...]
    logits = lax.dot_general(x, wg_ref[...], (((1,), (1,)), ((), ())),
                             preferred_element_type=jnp.float32)
    m = jnp.max(logits, axis=1, keepdims=True)
    ex = jnp.exp(logits - m)
    p = ex / jnp.sum(ex, axis=1, keepdims=True)
    b = p + bg_ref[...]

    lane = lax.broadcasted_iota(jnp.int32, b.shape, 1)
    m1 = jnp.max(b, axis=1, keepdims=True)
    i1 = jnp.min(jnp.where(b == m1, lane, _E), axis=1, keepdims=True)
    oh1 = lane == i1
    b2 = jnp.where(oh1, jnp.float32(-1e30), b)
    m2 = jnp.max(b2, axis=1, keepdims=True)
    i2 = jnp.min(jnp.where(b2 == m2, lane, _E), axis=1, keepdims=True)
    oh2 = lane == i2
    w1 = jnp.sum(jnp.where(oh1, p, 0.0), axis=1, keepdims=True)
    w2 = jnp.sum(jnp.where(oh2, p, 0.0), axis=1, keepdims=True)

    mm = oh1.astype(jnp.float32) + oh2.astype(jnp.float32)
    csum = lax.dot_general(ltri_ref[...], mm, (((1,), (0,)), ((), ())),
                           preferred_element_type=jnp.float32)
    baserow = base_ref[0:1, :]
    cb = csum + baserow
    r1 = jnp.sum(jnp.where(oh1, cb, 0.0), axis=1, keepdims=True)
    r2 = jnp.sum(jnp.where(oh2, cb, 0.0), axis=1, keepdims=True)

    v = jnp.concatenate(
        [r1, r2, i1.astype(jnp.float32), i2.astype(jnp.float32), w1, w2],
        axis=1)
    vt = lax.dot_general(v, eye_ref[...], (((0,), (0,)), ((), ())),
                         preferred_element_type=jnp.float32)
    route_ref[0] = vt

    newbase = baserow + jnp.sum(mm, axis=0, keepdims=True)
    base_ref[0:1, :] = newbase
    off_ref[...] = lax.dot_general(newbase, l64x_ref[...],
                                   (((1,), (1,)), ((), ())),
                                   preferred_element_type=jnp.float32)
    cum_ref[...] = lax.dot_general(newbase, l64i_ref[...],
                                   (((1,), (1,)), ((), ())),
                                   preferred_element_type=jnp.float32)


def _expert_ffn(x, w1, b1, w3, b3, w2, b2, c_exp):
    xc = x.astype(w1.dtype)
    s1 = lax.dot_general(xc, w1, (((1,), (1,)), ((), ())),
                         preferred_element_type=jnp.float32) + b1
    s1 = s1 * lax.logistic(s1)
    s3 = lax.dot_general(xc, w3, (((1,), (1,)), ((), ())),
                         preferred_element_type=jnp.float32) + b3
    xs = s1 * s3
    xt = jnp.sqrt(jnp.clip(
        jnp.sum(xs * xs, axis=1, keepdims=True) + c_exp, 1e-8))
    h = jnp.concatenate([xt, xs], axis=1).astype(w2.dtype)
    os_ = lax.dot_general(h, w2, (((1,), (1,)), ((), ())),
                          preferred_element_type=jnp.float32) + b2
    ot = jnp.sqrt(jnp.clip(
        jnp.sum(os_ * os_, axis=1, keepdims=True) + _C, 1e-8))
    return jnp.concatenate([ot, os_], axis=1)


def _grouped_body(tiles_ref, experts_ref, starts_ref, ends_ref, inits_ref,
                  xs_ref, w1_hbm, b1_ref, w3_hbm, b3_ref, w2_hbm, b2_ref,
                  ys_ref, w1buf, w3buf, w2buf, s1, s3, s2):
    g = pl.program_id(0)
    ng = pl.num_programs(0)
    slot = lax.rem(g, 2)
    nslot = lax.rem(g + 1, 2)

    def _start(e, sl):
        pltpu.make_async_copy(w1_hbm.at[:, e, :], w1buf.at[sl],
                              s1.at[sl]).start()
        pltpu.make_async_copy(w3_hbm.at[:, e, :], w3buf.at[sl],
                              s3.at[sl]).start()
        pltpu.make_async_copy(w2_hbm.at[:, e, :], w2buf.at[sl],
                              s2.at[sl]).start()

    @pl.when(g == 0)
    def _():
        _start(experts_ref[0], 0)

    @pl.when(g + 1 < ng)
    def _():
        _start(experts_ref[g + 1], nslot)

    e_g = experts_ref[g]
    pltpu.make_async_copy(w1_hbm.at[:, e_g, :], w1buf.at[slot],
                          s1.at[slot]).wait()
    pltpu.make_async_copy(w3_hbm.at[:, e_g, :], w3buf.at[slot],
                          s3.at[slot]).wait()
    pltpu.make_async_copy(w2_hbm.at[:, e_g, :], w2buf.at[slot],
                          s2.at[slot]).wait()

    ye = _expert_ffn(xs_ref[...], w1buf[slot], b1_ref[0], w3buf[slot],
                     b3_ref[0], w2buf[slot], b2_ref[0], _C)
    pos = _ROW_TILE * tiles_ref[g] + lax.broadcasted_iota(
        jnp.int32, (_ROW_TILE, 1), 0)
    mask = (pos >= starts_ref[g]) & (pos < ends_ref[g])

    @pl.when(inits_ref[g] == 1)
    def _():
        ys_ref[...] = jnp.where(mask, ye, 0.0)

    @pl.when(inits_ref[g] == 0)
    def _():
        ys_ref[...] = jnp.where(mask, ye, ys_ref[...])


def _dispatch_body(e_hbm, r_hbm, off_hbm, cum_hbm, x_hbm,
                   xs_hbm, dest_hbm, tiles_hbm, experts_hbm, starts_hbm,
                   ends_hbm, inits_hbm,
                   ev, rv, offs, cum, destv, xv,
                   first_v, nbase_v, pa, pb, t96, e96, s96, en96, in96, sem):
    wid = lax.axis_index("s") * 2 + lax.axis_index("c")
    base_j = wid * _PW
    tok0 = (wid % (_NW // 2)) * _PW

    pltpu.sync_copy(e_hbm.at[pl.ds(base_j, _PW)], ev)
    pltpu.sync_copy(r_hbm.at[pl.ds(base_j, _PW)], rv)
    pltpu.sync_copy(off_hbm, offs)
    pltpu.sync_copy(cum_hbm, cum)

    for c in range(_PW // 16):
        ej = ev[pl.ds(c * 16, 16)]
        oj = plsc.load_gather(offs, [ej])
        destv[pl.ds(c * 16, 16)] = oj + rv[pl.ds(c * 16, 16)]
    pltpu.sync_copy(destv, dest_hbm.at[pl.ds(base_j, _PW)])

    pltpu.sync_copy(x_hbm.at[pl.ds(tok0, _PW)], xv)
    pltpu.async_copy(xv, xs_hbm.at[destv], sem).wait()

    @pl.when(wid == 0)
    def _():
        for c in range(_N_TILES // 16):
            pvec = (lax.iota(jnp.int32, 16) + 16 * c) * _ROW_TILE
            fcnt = jnp.zeros((16,), jnp.int32)
            lcnt = jnp.zeros((16,), jnp.int32)
            for c2 in range(_E // 16):
                cchunk = cum[pl.ds(c2 * 16, 16)]
                for k in range(16):
                    ce = cchunk[k]
                    fcnt = fcnt + (ce <= pvec).astype(jnp.int32)
                    lcnt = lcnt + (
                        ce <= pvec + (_ROW_TILE - 1)).astype(jnp.int32)
            first_v[pl.ds(c * 16, 16)] = fcnt
            nbase_v[pl.ds(c * 16, 16)] = lcnt - fcnt + 1
        zeros16 = jnp.zeros((16,), jnp.int32)
        pa[pl.ds(0, 16)] = zeros16
        for c in range(_N_TILES // 16):
            pa[pl.ds(16 + c * 16, 16)] = nbase_v[pl.ds(c * 16, 16)]
        cur, nxt = pa, pb
        for kk in (1, 2, 4, 8, 16):
            nxt[pl.ds(0, 16)] = zeros16
            for c in range(_N_TILES // 16):
                nxt[pl.ds(16 + c * 16, 16)] = (
                    cur[pl.ds(16 + c * 16, 16)]
                    + cur[pl.ds(16 + c * 16 - kk, 16)])
            cur, nxt = nxt, cur
        lastchunk = cur[pl.ds(16 + _N_TILES - 16, 16)]
        tot_items = lastchunk[15]
        for c in range(_N_TILES // 16):
            inc = cur[pl.ds(16 + c * 16, 16)]
            ni = nbase_v[pl.ds(c * 16, 16)]
            nbase_v[pl.ds(c * 16, 16)] = inc - ni
        for c in range(_G_ITEMS // 16):
            gvec = lax.iota(jnp.int32, 16) + 16 * c
            tcnt = jnp.zeros((16,), jnp.int32)
            for c2 in range(_N_TILES // 16):
                bchunk = nbase_v[pl.ds(c2 * 16, 16)]
                for k in range(16):
                    tcnt = tcnt + (bchunk[k] <= gvec).astype(jnp.int32)
            tg = tcnt - 1
            bg_ = plsc.load_gather(nbase_v, [tg])
            fg = plsc.load_gather(first_v, [tg])
            eg = fg + gvec - bg_
            eg = jnp.minimum(jnp.maximum(eg, 0), _E - 1)
            og = plsc.load_gather(offs, [eg])
            cg = plsc.load_gather(cum, [eg])
            st = jnp.maximum(og, tg * _ROW_TILE)
            en = jnp.minimum(cg, tg * _ROW_TILE + _ROW_TILE)
            valid = gvec < tot_items
            st = jnp.where(valid, st, 0)
            en = jnp.where(valid, en, 0)
            init = (gvec == bg_).astype(jnp.int32)
            t96[pl.ds(c * 16, 16)] = tg
            e96[pl.ds(c * 16, 16)] = eg
            s96[pl.ds(c * 16, 16)] = st
            en96[pl.ds(c * 16, 16)] = en
            in96[pl.ds(c * 16, 16)] = init
        pltpu.sync_copy(t96, tiles_hbm)
        pltpu.sync_copy(e96, experts_hbm)
        pltpu.sync_copy(s96, starts_hbm)
        pltpu.sync_copy(en96, ends_hbm)
        pltpu.sync_copy(in96, inits_hbm)


def _sc_dispatch(e_flat, r_flat, offs_arr, cum_arr, x):
    f32, i32 = jnp.float32, jnp.int32
    mesh = plsc.VectorSubcoreMesh(core_axis_name="c", subcore_axis_name="s")
    call = pl.kernel(
        _dispatch_body,
        out_type=(
            jax.ShapeDtypeStruct((_NP, _DIM), f32),
            jax.ShapeDtypeStruct((_NP,), i32),
            jax.ShapeDtypeStruct((_G_ITEMS,), i32),
            jax.ShapeDtypeStruct((_G_ITEMS,), i32),
            jax.ShapeDtypeStruct((_G_ITEMS,), i32),
            jax.ShapeDtypeStruct((_G_ITEMS,), i32),
            jax.ShapeDtypeStruct((_G_ITEMS,), i32),
        ),
        mesh=mesh,
        scratch_types=[
            pltpu.VMEM((_PW,), i32),
            pltpu.VMEM((_PW,), i32),
            pltpu.VMEM((_E,), i32),
            pltpu.VMEM((_E,), i32),
            pltpu.VMEM((_PW,), i32),
            pltpu.VMEM((_PW, _DIM), f32),
            pltpu.VMEM((_N_TILES,), i32),
            pltpu.VMEM((_N_TILES,), i32),
            pltpu.VMEM((16 + _N_TILES,), i32),
            pltpu.VMEM((16 + _N_TILES,), i32),
            pltpu.VMEM((_G_ITEMS,), i32),
            pltpu.VMEM((_G_ITEMS,), i32),
            pltpu.VMEM((_G_ITEMS,), i32),
            pltpu.VMEM((_G_ITEMS,), i32),
            pltpu.VMEM((_G_ITEMS,), i32),
            pltpu.SemaphoreType.DMA,
        ],
        compiler_params=pltpu.CompilerParams(needs_layout_passes=False),
    )
    return call(e_flat, r_flat, offs_arr, cum_arr, x)


def _combine_body(ys_hbm, dest_hbm, w_hbm, acc_hbm,
                  d0, d1, wv0, wv1, rows0, rows1, accv, sem0, sem1):
    wid = lax.axis_index("s") * 2 + lax.axis_index("c")
    half = _TW // 2
    for bat in range(2):
        tbase = wid * _TW + bat * half
        pltpu.sync_copy(dest_hbm.at[pl.ds(tbase, half)], d0)
        pltpu.sync_copy(dest_hbm.at[pl.ds(_T + tbase, half)], d1)
        pltpu.sync_copy(w_hbm.at[pl.ds(tbase, half)], wv0)
        pltpu.sync_copy(w_hbm.at[pl.ds(_T + tbase, half)], wv1)
        c0 = pltpu.async_copy(ys_hbm.at[d0], rows0, sem0)
        c1 = pltpu.async_copy(ys_hbm.at[d1], rows1, sem1)
        c0.wait()
        c1.wait()

        def body(t, _):
            tsplat = jnp.full((16,), t, jnp.int32)
            w0 = plsc.load_gather(wv0, [tsplat])
            w1 = plsc.load_gather(wv1, [tsplat])
            for c in range(_DIM // 16):
                sl = pl.ds(c * 16, 16)
                accv[t, sl] = rows0[t, sl] * w0 + rows1[t, sl] * w1
            return 0

        lax.fori_loop(0, half, body, 0)
        pltpu.sync_copy(accv, acc_hbm.at[pl.ds(tbase, half)])


def _sc_combine(ys, dest, w_flat):
    f32, i32 = jnp.float32, jnp.int32
    half = _TW // 2
    mesh = plsc.VectorSubcoreMesh(core_axis_name="c", subcore_axis_name="s")
    call = pl.kernel(
        _combine_body,
        out_type=jax.ShapeDtypeStruct((_T, _DIM), f32),
        mesh=mesh,
        scratch_types=[
            pltpu.VMEM((half,), i32),
            pltpu.VMEM((half,), i32),
            pltpu.VMEM((half,), f32),
            pltpu.VMEM((half,), f32),
            pltpu.VMEM((half, _DIM), f32),
            pltpu.VMEM((half, _DIM), f32),
            pltpu.VMEM((half, _DIM), f32),
            pltpu.SemaphoreType.DMA,
            pltpu.SemaphoreType.DMA,
        ],
        compiler_params=pltpu.CompilerParams(needs_layout_passes=False),
    )
    return call(ys, dest, w_flat)


def _shared_body(x_ref, ws1_ref, bs1_ref, ws3_ref, bs3_ref, ws2_ref, bs2_ref,
                 out_ref):
    out_ref[...] = _expert_ffn(x_ref[...], ws1_ref[...], bs1_ref[...],
                               ws3_ref[...], bs3_ref[...], ws2_ref[...],
                               bs2_ref[...], _C)


def _midpoint_norm(z):
    lane = lax.broadcasted_iota(jnp.int32, z.shape, 1)
    sgn = jnp.where(lane == 0, -1.0, 1.0)
    inner = jnp.sum(z * z * sgn, axis=1, keepdims=True)
    return z / jnp.sqrt(jnp.clip(-inner, 1e-8))


def _final_body(shared_ref, acc_ref, out_ref):
    routed = _midpoint_norm(acc_ref[...])
    out_ref[...] = _midpoint_norm(shared_ref[...] + routed)


def kernel(x, Wg, bg, W1, b1, W3, b3, W2, b2, Ws1, bs1, Ws3, bs3, Ws2, bs2):
    f32, i32 = jnp.float32, jnp.int32
    wg_ext = jnp.concatenate([jnp.zeros((_E, 1), f32), Wg], axis=1)
    ltri = jnp.tri(_GATE_TILE, k=-1, dtype=f32)
    eye = jnp.eye(_GATE_TILE, dtype=f32)
    l64x = jnp.tri(_E, k=-1, dtype=f32)
    l64i = jnp.tri(_E, k=0, dtype=f32)

    route, offv, cumv = pl.pallas_call(
        _gate_body,
        grid=(_N_GATE,),
        in_specs=[
            pl.BlockSpec((_GATE_TILE, _DIM), lambda i: (i, 0)),
            pl.BlockSpec((_E, _DIM), lambda i: (0, 0)),
            pl.BlockSpec((1, _E), lambda i: (0, 0)),
            pl.BlockSpec((_GATE_TILE, _GATE_TILE), lambda i: (0, 0)),
            pl.BlockSpec((_GATE_TILE, _GATE_TILE), lambda i: (0, 0)),
            pl.BlockSpec((_E, _E), lambda i: (0, 0)),
            pl.BlockSpec((_E, _E), lambda i: (0, 0)),
        ],
        out_specs=[
            pl.BlockSpec((1, 6, _GATE_TILE), lambda i: (i, 0, 0)),
            pl.BlockSpec((1, _E), lambda i: (0, 0)),
            pl.BlockSpec((1, _E), lambda i: (0, 0)),
        ],
        out_shape=[
            jax.ShapeDtypeStruct((_N_GATE, 6, _GATE_TILE), f32),
            jax.ShapeDtypeStruct((1, _E), f32),
            jax.ShapeDtypeStruct((1, _E), f32),
        ],
        scratch_shapes=[pltpu.VMEM((8, _E), f32)],
    )(x, wg_ext, bg.reshape(1, _E), ltri, eye, l64x, l64i)

    r_flat = jnp.concatenate(
        [route[:, 0, :].reshape(-1), route[:, 1, :].reshape(-1)]).astype(i32)
    e_flat = jnp.concatenate(
        [route[:, 2, :].reshape(-1), route[:, 3, :].reshape(-1)]).astype(i32)
    w_flat = jnp.concatenate(
        [route[:, 4, :].reshape(-1), route[:, 5, :].reshape(-1)])
    offs_arr = offv.reshape(_E).astype(i32)
    cum_arr = cumv.reshape(_E).astype(i32)

    xs, dest, tiles, experts, starts, ends, inits = _sc_dispatch(
        e_flat, r_flat, offs_arr, cum_arr, x)

    def _native(w):
        return w.transpose(1, 0, 2)

    w1n = _native(W1)
    w3n = _native(W3)
    w2n = _native(W2)

    grid_spec = pltpu.PrefetchScalarGridSpec(
        num_scalar_prefetch=5,
        grid=(_G_ITEMS,),
        in_specs=[
            pl.BlockSpec((_ROW_TILE, _DIM),
                         lambda g, tr, er, sr, enr, ir: (tr[g], 0)),
            pl.BlockSpec(memory_space=pl.ANY),
            pl.BlockSpec((1, 1, _INTER - 1),
                         lambda g, tr, er, sr, enr, ir: (er[g], 0, 0)),
            pl.BlockSpec(memory_space=pl.ANY),
            pl.BlockSpec((1, 1, _INTER - 1),
                         lambda g, tr, er, sr, enr, ir: (er[g], 0, 0)),
            pl.BlockSpec(memory_space=pl.ANY),
            pl.BlockSpec((1, 1, _DIM - 1),
                         lambda g, tr, er, sr, enr, ir: (er[g], 0, 0)),
        ],
        out_specs=pl.BlockSpec((_ROW_TILE, _DIM),
                               lambda g, tr, er, sr, enr, ir: (tr[g], 0)),
        scratch_shapes=[
            pltpu.VMEM((2, _INTER - 1, _DIM), f32),
            pltpu.VMEM((2, _INTER - 1, _DIM), f32),
            pltpu.VMEM((2, _DIM - 1, _INTER), f32),
            pltpu.SemaphoreType.DMA((2,)),
            pltpu.SemaphoreType.DMA((2,)),
            pltpu.SemaphoreType.DMA((2,)),
        ],
    )
    ys = pl.pallas_call(
        _grouped_body,
        grid_spec=grid_spec,
        out_shape=jax.ShapeDtypeStruct((_NP, _DIM), f32),
    )(tiles, experts, starts, ends, inits, xs,
      w1n, b1.reshape(_E, 1, _INTER - 1), w3n, b3.reshape(_E, 1, _INTER - 1),
      w2n, b2.reshape(_E, 1, _DIM - 1))

    acc = _sc_combine(ys, dest, w_flat)

    shared = pl.pallas_call(
        _shared_body,
        grid=(_T // 128,),
        in_specs=[
            pl.BlockSpec((128, _DIM), lambda i: (i, 0)),
            pl.BlockSpec((_INTER - 1, _DIM), lambda i: (0, 0)),
            pl.BlockSpec((1, _INTER - 1), lambda i: (0, 0)),
            pl.BlockSpec((_INTER - 1, _DIM), lambda i: (0, 0)),
            pl.BlockSpec((1, _INTER - 1), lambda i: (0, 0)),
            pl.BlockSpec((_DIM - 1, _INTER), lambda i: (0, 0)),
            pl.BlockSpec((1, _DIM - 1), lambda i: (0, 0)),
        ],
        out_specs=pl.BlockSpec((128, _DIM), lambda i: (i, 0)),
        out_shape=jax.ShapeDtypeStruct((_T, _DIM), f32),
    )(x, Ws1, bs1.reshape(1, _INTER - 1), Ws3, bs3.reshape(1, _INTER - 1),
      Ws2, bs2.reshape(1, _DIM - 1))

    out = pl.pallas_call(
        _final_body,
        grid=(_T // 128,),
        in_specs=[
            pl.BlockSpec((128, _DIM), lambda i: (i, 0)),
            pl.BlockSpec((128, _DIM), lambda i: (i, 0)),
        ],
        out_specs=pl.BlockSpec((128, _DIM), lambda i: (i, 0)),
        out_shape=jax.ShapeDtypeStruct((_T, _DIM), f32),
    )(shared, acc)
    return out


# speedup vs baseline: 1.8072x; 1.0019x over previous
import functools

import jax
import jax.numpy as jnp
from jax import lax
from jax.experimental import pallas as pl
from jax.experimental.pallas import tpu as pltpu
from jax.experimental.pallas import tpu_sc as plsc
from jax.experimental.layout import Format, Layout, with_layout_constraint

_DIM = 768
_INTER = 256
_E = 64
_T = 2048
_C = 1.0

_GATE_TILE = 256
_N_GATE = _T // _GATE_TILE
_NP = 2 * _T
_ROW_TILE = 128
_N_TILES = _NP // _ROW_TILE
_G_ITEMS = _N_TILES + _E
_NW = 32
_PW = _NP // _NW
_TW = _T // _NW


def _gate_body(x_ref, wg_ref, bg_ref, ltri_ref, eye_ref, l64x_ref, l64i_ref,
               route_ref, off_ref, cum_ref, base_ref):
    i = pl.program_id(0)

    @pl.when(i == 0)
    def _():
        base_ref[...] = jnp.zeros_like(base_ref)

    x = x_ref[...]
    logits = lax.dot_general(x, wg_ref[...], (((1,), (1,)), ((), ())),
                             preferred_element_type=jnp.float32)
    m = jnp.max(logits, axis=1, keepdims=True)
    ex = jnp.exp(logits - m)
    p = ex / jnp.sum(ex, axis=1, keepdims=True)
    b = p + bg_ref[...]

    lane = lax.broadcasted_iota(jnp.int32, b.shape, 1)
    m1 = jnp.max(b, axis=1, keepdims=True)
    i1 = jnp.min(jnp.where(b == m1, lane, _E), axis=1, keepdims=True)
    oh1 = lane == i1
    b2 = jnp.where(oh1, jnp.float32(-1e30), b)
    m2 = jnp.max(b2, axis=1, keepdims=True)
    i2 = jnp.min(jnp.where(b2 == m2, lane, _E), axis=1, keepdims=True)
    oh2 = lane == i2
    w1 = jnp.sum(jnp.where(oh1, p, 0.0), axis=1, keepdims=True)
    w2 = jnp.sum(jnp.where(oh2, p, 0.0), axis=1, keepdims=True)

    mm = oh1.astype(jnp.float32) + oh2.astype(jnp.float32)
    csum = lax.dot_general(ltri_ref[...], mm, (((1,), (0,)), ((), ())),
                           preferred_element_type=jnp.float32)
    baserow = base_ref[0:1, :]
    cb = csum + baserow
    r1 = jnp.sum(jnp.where(oh1, cb, 0.0), axis=1, keepdims=True)
    r2 = jnp.sum(jnp.where(oh2, cb, 0.0), axis=1, keepdims=True)

    v = jnp.concatenate(
        [r1, r2, i1.astype(jnp.float32), i2.astype(jnp.float32), w1, w2],
        axis=1)
    vt = lax.dot_general(v, eye_ref[...], (((0,), (0,)), ((), ())),
                         preferred_element_type=jnp.float32)
    route_ref[0] = vt

    newbase = baserow + jnp.sum(mm, axis=0, keepdims=True)
    base_ref[0:1, :] = newbase
    off_ref[...] = lax.dot_general(newbase, l64x_ref[...],
                                   (((1,), (1,)), ((), ())),
                                   preferred_element_type=jnp.float32)
    cum_ref[...] = lax.dot_general(newbase, l64i_ref[...],
                                   (((1,), (1,)), ((), ())),
                                   preferred_element_type=jnp.float32)


def _expert_ffn(x, w1, b1, w3, b3, w2, b2, c_exp):
    xc = x.astype(w1.dtype)
    s1 = lax.dot_general(xc, w1, (((1,), (1,)), ((), ())),
                         preferred_element_type=jnp.float32) + b1
    s1 = s1 * lax.logistic(s1)
    s3 = lax.dot_general(xc, w3, (((1,), (1,)), ((), ())),
                         preferred_element_type=jnp.float32) + b3
    xs = s1 * s3
    xt = jnp.sqrt(jnp.clip(
        jnp.sum(xs * xs, axis=1, keepdims=True) + c_exp, 1e-8))
    h = jnp.concatenate([xt, xs], axis=1).astype(w2.dtype)
    os_ = lax.dot_general(h, w2, (((1,), (1,)), ((), ())),
                          preferred_element_type=jnp.float32) + b2
    ot = jnp.sqrt(jnp.clip(
        jnp.sum(os_ * os_, axis=1, keepdims=True) + _C, 1e-8))
    return jnp.concatenate([ot, os_], axis=1)


def _grouped_body(tiles_ref, experts_ref, starts_ref, ends_ref, inits_ref,
                  fetch_ref, slots_ref, xs_ref, w1_hbm, b1_ref, w3_hbm,
                  b3_ref, w2_hbm, b2_ref, ys_ref, w1buf, w3buf, w2buf,
                  s1, s3, s2):
    g = pl.program_id(0)
    ng = pl.num_programs(0)
    slot = slots_ref[g]

    def _start(e, sl):
        pltpu.make_async_copy(w1_hbm.at[:, e, :], w1buf.at[sl],
                              s1.at[sl]).start()
        pltpu.make_async_copy(w3_hbm.at[:, e, :], w3buf.at[sl],
                              s3.at[sl]).start()
        pltpu.make_async_copy(w2_hbm.at[:, e, :], w2buf.at[sl],
                              s2.at[sl]).start()

    @pl.when(g == 0)
    def _():
        _start(experts_ref[0], slots_ref[0])

    @pl.when((g + 1 < ng) & (fetch_ref[jnp.minimum(g + 1, ng - 1)] == 1))
    def _():
        _start(experts_ref[g + 1], slots_ref[g + 1])

    e_g = experts_ref[g]

    @pl.when(fetch_ref[g] == 1)
    def _():
        pltpu.make_async_copy(w1_hbm.at[:, e_g, :], w1buf.at[slot],
                              s1.at[slot]).wait()
        pltpu.make_async_copy(w3_hbm.at[:, e_g, :], w3buf.at[slot],
                              s3.at[slot]).wait()
        pltpu.make_async_copy(w2_hbm.at[:, e_g, :], w2buf.at[slot],
                              s2.at[slot]).wait()

    ye = _expert_ffn(xs_ref[...], w1buf[slot], b1_ref[0], w3buf[slot],
                     b3_ref[0], w2buf[slot], b2_ref[0], _C)
    pos = _ROW_TILE * tiles_ref[g] + lax.broadcasted_iota(
        jnp.int32, (_ROW_TILE, 1), 0)
    mask = (pos >= starts_ref[g]) & (pos < ends_ref[g])

    @pl.when(inits_ref[g] == 1)
    def _():
        ys_ref[...] = jnp.where(mask, ye, 0.0)

    @pl.when(inits_ref[g] == 0)
    def _():
        ys_ref[...] = jnp.where(mask, ye, ys_ref[...])


def _dispatch_body(e_hbm, r_hbm, off_hbm, cum_hbm, x_hbm,
                   xs_hbm, dest_hbm, tiles_hbm, experts_hbm, starts_hbm,
                   ends_hbm, inits_hbm,
                   ev, rv, offs, cum, destv, xv,
                   first_v, nbase_v, pa, pb, t96, e96, s96, en96, in96, sem):
    wid = lax.axis_index("s") * 2 + lax.axis_index("c")
    base_j = wid * _PW
    tok0 = (wid % (_NW // 2)) * _PW

    pltpu.sync_copy(e_hbm.at[pl.ds(base_j, _PW)], ev)
    pltpu.sync_copy(r_hbm.at[pl.ds(base_j, _PW)], rv)
    pltpu.sync_copy(off_hbm, offs)
    pltpu.sync_copy(cum_hbm, cum)

    for c in range(_PW // 16):
        ej = ev[pl.ds(c * 16, 16)]
        oj = plsc.load_gather(offs, [ej])
        destv[pl.ds(c * 16, 16)] = oj + rv[pl.ds(c * 16, 16)]
    pltpu.sync_copy(destv, dest_hbm.at[pl.ds(base_j, _PW)])

    pltpu.sync_copy(x_hbm.at[pl.ds(tok0, _PW)], xv)
    pltpu.async_copy(xv, xs_hbm.at[destv], sem).wait()

    @pl.when(wid == 0)
    def _():
        for c in range(_N_TILES // 16):
            pvec = (lax.iota(jnp.int32, 16) + 16 * c) * _ROW_TILE
            fcnt = jnp.zeros((16,), jnp.int32)
            lcnt = jnp.zeros((16,), jnp.int32)
            for c2 in range(_E // 16):
                cchunk = cum[pl.ds(c2 * 16, 16)]
                for k in range(16):
                    ce = cchunk[k]
                    fcnt = fcnt + (ce <= pvec).astype(jnp.int32)
                    lcnt = lcnt + (
                        ce <= pvec + (_ROW_TILE - 1)).astype(jnp.int32)
            first_v[pl.ds(c * 16, 16)] = fcnt
            nbase_v[pl.ds(c * 16, 16)] = lcnt - fcnt + 1
        zeros16 = jnp.zeros((16,), jnp.int32)
        pa[pl.ds(0, 16)] = zeros16
        for c in range(_N_TILES // 16):
            pa[pl.ds(16 + c * 16, 16)] = nbase_v[pl.ds(c * 16, 16)]
        cur, nxt = pa, pb
        for kk in (1, 2, 4, 8, 16):
            nxt[pl.ds(0, 16)] = zeros16
            for c in range(_N_TILES // 16):
                nxt[pl.ds(16 + c * 16, 16)] = (
                    cur[pl.ds(16 + c * 16, 16)]
                    + cur[pl.ds(16 + c * 16 - kk, 16)])
            cur, nxt = nxt, cur
        lastchunk = cur[pl.ds(16 + _N_TILES - 16, 16)]
        tot_items = lastchunk[15]
        for c in range(_N_TILES // 16):
            inc = cur[pl.ds(16 + c * 16, 16)]
            ni = nbase_v[pl.ds(c * 16, 16)]
            nbase_v[pl.ds(c * 16, 16)] = inc - ni
        for c in range(_G_ITEMS // 16):
            gvec = lax.iota(jnp.int32, 16) + 16 * c
            tcnt = jnp.zeros((16,), jnp.int32)
            for c2 in range(_N_TILES // 16):
                bchunk = nbase_v[pl.ds(c2 * 16, 16)]
                for k in range(16):
                    tcnt = tcnt + (bchunk[k] <= gvec).astype(jnp.int32)
            tg = tcnt - 1
            bg_ = plsc.load_gather(nbase_v, [tg])
            fg = plsc.load_gather(first_v, [tg])
            eg = fg + gvec - bg_
            eg = jnp.minimum(jnp.maximum(eg, 0), _E - 1)
            og = plsc.load_gather(offs, [eg])
            cg = plsc.load_gather(cum, [eg])
            st = jnp.maximum(og, tg * _ROW_TILE)
            en = jnp.minimum(cg, tg * _ROW_TILE + _ROW_TILE)
            valid = gvec < tot_items
            st = jnp.where(valid, st, 0)
            en = jnp.where(valid, en, 0)
            init = (gvec == bg_).astype(jnp.int32)
            t96[pl.ds(c * 16, 16)] = tg
            e96[pl.ds(c * 16, 16)] = eg
            s96[pl.ds(c * 16, 16)] = st
            en96[pl.ds(c * 16, 16)] = en
            in96[pl.ds(c * 16, 16)] = init
        pltpu.sync_copy(t96, tiles_hbm)
        pltpu.sync_copy(e96, experts_hbm)
        pltpu.sync_copy(s96, starts_hbm)
        pltpu.sync_copy(en96, ends_hbm)
        pltpu.sync_copy(in96, inits_hbm)


def _sc_dispatch(e_flat, r_flat, offs_arr, cum_arr, x):
    f32, i32 = jnp.float32, jnp.int32
    mesh = plsc.VectorSubcoreMesh(core_axis_name="c", subcore_axis_name="s")
    call = pl.kernel(
        _dispatch_body,
        out_type=(
            jax.ShapeDtypeStruct((_NP, _DIM), f32),
            jax.ShapeDtypeStruct((_NP,), i32),
            jax.ShapeDtypeStruct((_G_ITEMS,), i32),
            jax.ShapeDtypeStruct((_G_ITEMS,), i32),
            jax.ShapeDtypeStruct((_G_ITEMS,), i32),
            jax.ShapeDtypeStruct((_G_ITEMS,), i32),
            jax.ShapeDtypeStruct((_G_ITEMS,), i32),
        ),
        mesh=mesh,
        scratch_types=[
            pltpu.VMEM((_PW,), i32),
            pltpu.VMEM((_PW,), i32),
            pltpu.VMEM((_E,), i32),
            pltpu.VMEM((_E,), i32),
            pltpu.VMEM((_PW,), i32),
            pltpu.VMEM((_PW, _DIM), f32),
            pltpu.VMEM((_N_TILES,), i32),
            pltpu.VMEM((_N_TILES,), i32),
            pltpu.VMEM((16 + _N_TILES,), i32),
            pltpu.VMEM((16 + _N_TILES,), i32),
            pltpu.VMEM((_G_ITEMS,), i32),
            pltpu.VMEM((_G_ITEMS,), i32),
            pltpu.VMEM((_G_ITEMS,), i32),
            pltpu.VMEM((_G_ITEMS,), i32),
            pltpu.VMEM((_G_ITEMS,), i32),
            pltpu.SemaphoreType.DMA,
        ],
        compiler_params=pltpu.CompilerParams(needs_layout_passes=False),
    )
    return call(e_flat, r_flat, offs_arr, cum_arr, x)


def _combine_body(ys_hbm, dest_hbm, w_hbm, acc_hbm,
                  d0, d1, wv0, wv1, rows0, rows1, accv, sem0, sem1):
    wid = lax.axis_index("s") * 2 + lax.axis_index("c")
    half = _TW // 2
    for bat in range(2):
        tbase = wid * _TW + bat * half
        pltpu.sync_copy(dest_hbm.at[pl.ds(tbase, half)], d0)
        pltpu.sync_copy(dest_hbm.at[pl.ds(_T + tbase, half)], d1)
        pltpu.sync_copy(w_hbm.at[pl.ds(tbase, half)], wv0)
        pltpu.sync_copy(w_hbm.at[pl.ds(_T + tbase, half)], wv1)
        c0 = pltpu.async_copy(ys_hbm.at[d0], rows0, sem0)
        c1 = pltpu.async_copy(ys_hbm.at[d1], rows1, sem1)
        c0.wait()
        c1.wait()

        def body(t, _):
            tsplat = jnp.full((16,), t, jnp.int32)
            w0 = plsc.load_gather(wv0, [tsplat])
            w1 = plsc.load_gather(wv1, [tsplat])
            for c in range(_DIM // 16):
                sl = pl.ds(c * 16, 16)
                accv[t, sl] = rows0[t, sl] * w0 + rows1[t, sl] * w1
            return 0

        lax.fori_loop(0, half, body, 0)
        pltpu.sync_copy(accv, acc_hbm.at[pl.ds(tbase, half)])


def _sc_combine(ys, dest, w_flat):
    f32, i32 = jnp.float32, jnp.int32
    half = _TW // 2
    mesh = plsc.VectorSubcoreMesh(core_axis_name="c", subcore_axis_name="s")
    call = pl.kernel(
        _combine_body,
        out_type=jax.ShapeDtypeStruct((_T, _DIM), f32),
        mesh=mesh,
        scratch_types=[
            pltpu.VMEM((half,), i32),
            pltpu.VMEM((half,), i32),
            pltpu.VMEM((half,), f32),
            pltpu.VMEM((half,), f32),
            pltpu.VMEM((half, _DIM), f32),
            pltpu.VMEM((half, _DIM), f32),
            pltpu.VMEM((half, _DIM), f32),
            pltpu.SemaphoreType.DMA,
            pltpu.SemaphoreType.DMA,
        ],
        compiler_params=pltpu.CompilerParams(needs_layout_passes=False),
    )
    return call(ys, dest, w_flat)


def _shared_body(x_ref, ws1_ref, bs1_ref, ws3_ref, bs3_ref, ws2_ref, bs2_ref,
                 out_ref):
    out_ref[...] = _expert_ffn(x_ref[...], ws1_ref[...], bs1_ref[...],
                               ws3_ref[...], bs3_ref[...], ws2_ref[...],
                               bs2_ref[...], _C)


def _midpoint_norm(z):
    lane = lax.broadcasted_iota(jnp.int32, z.shape, 1)
    sgn = jnp.where(lane == 0, -1.0, 1.0)
    inner = jnp.sum(z * z * sgn, axis=1, keepdims=True)
    return z / jnp.sqrt(jnp.clip(-inner, 1e-8))


def _final_body(shared_ref, acc_ref, out_ref):
    routed = _midpoint_norm(acc_ref[...])
    out_ref[...] = _midpoint_norm(shared_ref[...] + routed)


def kernel(x, Wg, bg, W1, b1, W3, b3, W2, b2, Ws1, bs1, Ws3, bs3, Ws2, bs2):
    f32, i32 = jnp.float32, jnp.int32
    wg_ext = jnp.concatenate([jnp.zeros((_E, 1), f32), Wg], axis=1)
    ltri = jnp.tri(_GATE_TILE, k=-1, dtype=f32)
    eye = jnp.eye(_GATE_TILE, dtype=f32)
    l64x = jnp.tri(_E, k=-1, dtype=f32)
    l64i = jnp.tri(_E, k=0, dtype=f32)

    route, offv, cumv = pl.pallas_call(
        _gate_body,
        grid=(_N_GATE,),
        in_specs=[
            pl.BlockSpec((_GATE_TILE, _DIM), lambda i: (i, 0)),
            pl.BlockSpec((_E, _DIM), lambda i: (0, 0)),
            pl.BlockSpec((1, _E), lambda i: (0, 0)),
            pl.BlockSpec((_GATE_TILE, _GATE_TILE), lambda i: (0, 0)),
            pl.BlockSpec((_GATE_TILE, _GATE_TILE), lambda i: (0, 0)),
            pl.BlockSpec((_E, _E), lambda i: (0, 0)),
            pl.BlockSpec((_E, _E), lambda i: (0, 0)),
        ],
        out_specs=[
            pl.BlockSpec((1, 6, _GATE_TILE), lambda i: (i, 0, 0)),
            pl.BlockSpec((1, _E), lambda i: (0, 0)),
            pl.BlockSpec((1, _E), lambda i: (0, 0)),
        ],
        out_shape=[
            jax.ShapeDtypeStruct((_N_GATE, 6, _GATE_TILE), f32),
            jax.ShapeDtypeStruct((1, _E), f32),
            jax.ShapeDtypeStruct((1, _E), f32),
        ],
        scratch_shapes=[pltpu.VMEM((8, _E), f32)],
    )(x, wg_ext, bg.reshape(1, _E), ltri, eye, l64x, l64i)

    r_flat = jnp.concatenate(
        [route[:, 0, :].reshape(-1), route[:, 1, :].reshape(-1)]).astype(i32)
    e_flat = jnp.concatenate(
        [route[:, 2, :].reshape(-1), route[:, 3, :].reshape(-1)]).astype(i32)
    w_flat = jnp.concatenate(
        [route[:, 4, :].reshape(-1), route[:, 5, :].reshape(-1)])
    offs_arr = offv.reshape(_E).astype(i32)
    cum_arr = cumv.reshape(_E).astype(i32)

    xs, dest, tiles, experts, starts, ends, inits = _sc_dispatch(
        e_flat, r_flat, offs_arr, cum_arr, x)

    def _native(w):
        return w.transpose(1, 0, 2)

    w1n = _native(W1)
    w3n = _native(W3)
    w2n = _native(W2)

    fetch = jnp.concatenate(
        [jnp.ones((1,), i32), (experts[1:] != experts[:-1]).astype(i32)])
    slots = jnp.cumsum(fetch) % 2

    grid_spec = pltpu.PrefetchScalarGridSpec(
        num_scalar_prefetch=7,
        grid=(_G_ITEMS,),
        in_specs=[
            pl.BlockSpec((_ROW_TILE, _DIM),
                         lambda g, tr, er, sr, enr, ir, fr, sl: (tr[g], 0)),
            pl.BlockSpec(memory_space=pl.ANY),
            pl.BlockSpec((1, 1, _INTER - 1),
                         lambda g, tr, er, sr, enr, ir, fr, sl:
                         (er[g], 0, 0)),
            pl.BlockSpec(memory_space=pl.ANY),
            pl.BlockSpec((1, 1, _INTER - 1),
                         lambda g, tr, er, sr, enr, ir, fr, sl:
                         (er[g], 0, 0)),
            pl.BlockSpec(memory_space=pl.ANY),
            pl.BlockSpec((1, 1, _DIM - 1),
                         lambda g, tr, er, sr, enr, ir, fr, sl:
                         (er[g], 0, 0)),
        ],
        out_specs=pl.BlockSpec((_ROW_TILE, _DIM),
                               lambda g, tr, er, sr, enr, ir, fr, sl:
                               (tr[g], 0)),
        scratch_shapes=[
            pltpu.VMEM((2, _INTER - 1, _DIM), f32),
            pltpu.VMEM((2, _INTER - 1, _DIM), f32),
            pltpu.VMEM((2, _DIM - 1, _INTER), f32),
            pltpu.SemaphoreType.DMA((2,)),
            pltpu.SemaphoreType.DMA((2,)),
            pltpu.SemaphoreType.DMA((2,)),
        ],
    )
    ys = pl.pallas_call(
        _grouped_body,
        grid_spec=grid_spec,
        out_shape=jax.ShapeDtypeStruct((_NP, _DIM), f32),
    )(tiles, experts, starts, ends, inits, fetch, slots, xs,
      w1n, b1.reshape(_E, 1, _INTER - 1), w3n, b3.reshape(_E, 1, _INTER - 1),
      w2n, b2.reshape(_E, 1, _DIM - 1))

    acc = _sc_combine(ys, dest, w_flat)

    shared = pl.pallas_call(
        _shared_body,
        grid=(_T // 128,),
        in_specs=[
            pl.BlockSpec((128, _DIM), lambda i: (i, 0)),
            pl.BlockSpec((_INTER - 1, _DIM), lambda i: (0, 0)),
            pl.BlockSpec((1, _INTER - 1), lambda i: (0, 0)),
            pl.BlockSpec((_INTER - 1, _DIM), lambda i: (0, 0)),
            pl.BlockSpec((1, _INTER - 1), lambda i: (0, 0)),
            pl.BlockSpec((_DIM - 1, _INTER), lambda i: (0, 0)),
            pl.BlockSpec((1, _DIM - 1), lambda i: (0, 0)),
        ],
        out_specs=pl.BlockSpec((128, _DIM), lambda i: (i, 0)),
        out_shape=jax.ShapeDtypeStruct((_T, _DIM), f32),
    )(x, Ws1, bs1.reshape(1, _INTER - 1), Ws3, bs3.reshape(1, _INTER - 1),
      Ws2, bs2.reshape(1, _DIM - 1))

    out = pl.pallas_call(
        _final_body,
        grid=(_T // 128,),
        in_specs=[
            pl.BlockSpec((128, _DIM), lambda i: (i, 0)),
            pl.BlockSpec((128, _DIM), lambda i: (i, 0)),
        ],
        out_specs=pl.BlockSpec((128, _DIM), lambda i: (i, 0)),
        out_shape=jax.ShapeDtypeStruct((_T, _DIM), f32),
    )(shared, acc)
    return out


# speedup vs baseline: 1.8085x; 1.0007x over previous
import jax
import jax.numpy as jnp
from jax import lax
from jax.experimental import pallas as pl
from jax.experimental.pallas import tpu as pltpu
from jax.experimental.pallas import tpu_sc as plsc

_DIM = 768
_INTER = 256
_E = 64
_T = 2048
_C = 1.0

_GATE_TILE = 256
_N_GATE = _T // _GATE_TILE
_NP = 2 * _T
_ROW_TILE = 128
_N_TILES = _NP // _ROW_TILE
_G_ITEMS = _N_TILES + _E
_NW = 32
_PW = _NP // _NW
_TW = _T // _NW


def _gate_body(x_ref, wg_ref, bg_ref, ltri_ref, eye_ref, l64x_ref, l64i_ref,
               route_ref, off_ref, cum_ref, base_ref):
    i = pl.program_id(0)

    @pl.when(i == 0)
    def _():
        base_ref[...] = jnp.zeros_like(base_ref)

    x = x_ref[...]
    logits = lax.dot_general(x, wg_ref[...], (((1,), (1,)), ((), ())),
                             preferred_element_type=jnp.float32)
    m = jnp.max(logits, axis=1, keepdims=True)
    ex = jnp.exp(logits - m)
    p = ex / jnp.sum(ex, axis=1, keepdims=True)
    b = p + bg_ref[...]

    lane = lax.broadcasted_iota(jnp.int32, b.shape, 1)
    m1 = jnp.max(b, axis=1, keepdims=True)
    i1 = jnp.min(jnp.where(b == m1, lane, _E), axis=1, keepdims=True)
    oh1 = lane == i1
    b2 = jnp.where(oh1, jnp.float32(-1e30), b)
    m2 = jnp.max(b2, axis=1, keepdims=True)
    i2 = jnp.min(jnp.where(b2 == m2, lane, _E), axis=1, keepdims=True)
    oh2 = lane == i2
    w1 = jnp.sum(jnp.where(oh1, p, 0.0), axis=1, keepdims=True)
    w2 = jnp.sum(jnp.where(oh2, p, 0.0), axis=1, keepdims=True)

    mm = oh1.astype(jnp.float32) + oh2.astype(jnp.float32)
    csum = lax.dot_general(ltri_ref[...], mm, (((1,), (0,)), ((), ())),
                           preferred_element_type=jnp.float32)
    baserow = base_ref[0:1, :]
    cb = csum + baserow
    r1 = jnp.sum(jnp.where(oh1, cb, 0.0), axis=1, keepdims=True)
    r2 = jnp.sum(jnp.where(oh2, cb, 0.0), axis=1, keepdims=True)

    v = jnp.concatenate(
        [r1, r2, i1.astype(jnp.float32), i2.astype(jnp.float32), w1, w2],
        axis=1)
    vt = lax.dot_general(v, eye_ref[...], (((0,), (0,)), ((), ())),
                         preferred_element_type=jnp.float32)
    route_ref[0] = vt

    newbase = baserow + jnp.sum(mm, axis=0, keepdims=True)
    base_ref[0:1, :] = newbase
    off_ref[...] = lax.dot_general(newbase, l64x_ref[...],
                                   (((1,), (1,)), ((), ())),
                                   preferred_element_type=jnp.float32)
    cum_ref[...] = lax.dot_general(newbase, l64i_ref[...],
                                   (((1,), (1,)), ((), ())),
                                   preferred_element_type=jnp.float32)


def _expert_ffn(x, w1, b1, w3, b3, w2, b2, c_exp):
    xc = x.astype(w1.dtype)
    s1 = lax.dot_general(xc, w1, (((1,), (1,)), ((), ())),
                         preferred_element_type=jnp.float32) + b1
    s1 = s1 * lax.logistic(s1)
    s3 = lax.dot_general(xc, w3, (((1,), (1,)), ((), ())),
                         preferred_element_type=jnp.float32) + b3
    xs = s1 * s3
    xt = jnp.sqrt(jnp.clip(
        jnp.sum(xs * xs, axis=1, keepdims=True) + c_exp, 1e-8))
    h = jnp.concatenate([xt, xs], axis=1).astype(w2.dtype)
    os_ = lax.dot_general(h, w2, (((1,), (1,)), ((), ())),
                          preferred_element_type=jnp.float32) + b2
    ot = jnp.sqrt(jnp.clip(
        jnp.sum(os_ * os_, axis=1, keepdims=True) + _C, 1e-8))
    return jnp.concatenate([ot, os_], axis=1)


def _grouped_body(tiles_ref, experts_ref, starts_ref, ends_ref, inits_ref,
                  fetch_ref, slots_ref, xs_ref, w1_hbm, b1_ref, w3_hbm,
                  b3_ref, w2_hbm, b2_ref, ys_ref, w1buf, w3buf, w2buf,
                  s1, s3, s2):
    g = pl.program_id(0)
    ng = pl.num_programs(0)
    slot = slots_ref[g]

    def _start(e, sl):
        pltpu.make_async_copy(w1_hbm.at[:, e, :], w1buf.at[sl],
                              s1.at[sl]).start()
        pltpu.make_async_copy(w3_hbm.at[:, e, :], w3buf.at[sl],
                              s3.at[sl]).start()
        pltpu.make_async_copy(w2_hbm.at[:, e, :], w2buf.at[sl],
                              s2.at[sl]).start()

    @pl.when(g == 0)
    def _():
        _start(experts_ref[0], slots_ref[0])

    @pl.when((g + 1 < ng) & (fetch_ref[jnp.minimum(g + 1, ng - 1)] == 1))
    def _():
        _start(experts_ref[g + 1], slots_ref[g + 1])

    e_g = experts_ref[g]

    @pl.when(fetch_ref[g] == 1)
    def _():
        pltpu.make_async_copy(w1_hbm.at[:, e_g, :], w1buf.at[slot],
                              s1.at[slot]).wait()
        pltpu.make_async_copy(w3_hbm.at[:, e_g, :], w3buf.at[slot],
                              s3.at[slot]).wait()
        pltpu.make_async_copy(w2_hbm.at[:, e_g, :], w2buf.at[slot],
                              s2.at[slot]).wait()

    ye = _expert_ffn(xs_ref[...], w1buf[slot], b1_ref[0], w3buf[slot],
                     b3_ref[0], w2buf[slot], b2_ref[0], _C)
    pos = _ROW_TILE * tiles_ref[g] + lax.broadcasted_iota(
        jnp.int32, (_ROW_TILE, 1), 0)
    mask = (pos >= starts_ref[g]) & (pos < ends_ref[g])

    @pl.when(inits_ref[g] == 1)
    def _():
        ys_ref[...] = jnp.where(mask, ye, 0.0)

    @pl.when(inits_ref[g] == 0)
    def _():
        ys_ref[...] = jnp.where(mask, ye, ys_ref[...])


def _dispatch_body(e_hbm, r_hbm, off_hbm, cum_hbm, x_hbm,
                   xs_hbm, dest_hbm, tiles_hbm, experts_hbm, starts_hbm,
                   ends_hbm, inits_hbm,
                   ev, rv, offs, cum, destv, xv,
                   first_v, nbase_v, pa, pb, t96, e96, s96, en96, in96, sem):
    wid = lax.axis_index("s") * 2 + lax.axis_index("c")
    base_j = wid * _PW
    tok0 = (wid % (_NW // 2)) * _PW

    pltpu.sync_copy(e_hbm.at[pl.ds(base_j, _PW)], ev)
    pltpu.sync_copy(r_hbm.at[pl.ds(base_j, _PW)], rv)
    pltpu.sync_copy(off_hbm, offs)
    pltpu.sync_copy(cum_hbm, cum)

    for c in range(_PW // 16):
        ej = ev[pl.ds(c * 16, 16)]
        oj = plsc.load_gather(offs, [ej])
        destv[pl.ds(c * 16, 16)] = oj + rv[pl.ds(c * 16, 16)]
    pltpu.sync_copy(destv, dest_hbm.at[pl.ds(base_j, _PW)])

    pltpu.sync_copy(x_hbm.at[pl.ds(tok0, _PW)], xv)
    pltpu.async_copy(xv, xs_hbm.at[destv], sem).wait()

    @pl.when(wid == 0)
    def _():
        for c in range(_N_TILES // 16):
            pvec = (lax.iota(jnp.int32, 16) + 16 * c) * _ROW_TILE
            fcnt = jnp.zeros((16,), jnp.int32)
            lcnt = jnp.zeros((16,), jnp.int32)
            for c2 in range(_E // 16):
                cchunk = cum[pl.ds(c2 * 16, 16)]
                for k in range(16):
                    ce = cchunk[k]
                    fcnt = fcnt + (ce <= pvec).astype(jnp.int32)
                    lcnt = lcnt + (
                        ce <= pvec + (_ROW_TILE - 1)).astype(jnp.int32)
            first_v[pl.ds(c * 16, 16)] = fcnt
            nbase_v[pl.ds(c * 16, 16)] = lcnt - fcnt + 1
        zeros16 = jnp.zeros((16,), jnp.int32)
        pa[pl.ds(0, 16)] = zeros16
        for c in range(_N_TILES // 16):
            pa[pl.ds(16 + c * 16, 16)] = nbase_v[pl.ds(c * 16, 16)]
        cur, nxt = pa, pb
        for kk in (1, 2, 4, 8, 16):
            nxt[pl.ds(0, 16)] = zeros16
            for c in range(_N_TILES // 16):
                nxt[pl.ds(16 + c * 16, 16)] = (
                    cur[pl.ds(16 + c * 16, 16)]
                    + cur[pl.ds(16 + c * 16 - kk, 16)])
            cur, nxt = nxt, cur
        lastchunk = cur[pl.ds(16 + _N_TILES - 16, 16)]
        tot_items = lastchunk[15]
        for c in range(_N_TILES // 16):
            inc = cur[pl.ds(16 + c * 16, 16)]
            ni = nbase_v[pl.ds(c * 16, 16)]
            nbase_v[pl.ds(c * 16, 16)] = inc - ni
        for c in range(_G_ITEMS // 16):
            gvec = lax.iota(jnp.int32, 16) + 16 * c
            tcnt = jnp.zeros((16,), jnp.int32)
            for c2 in range(_N_TILES // 16):
                bchunk = nbase_v[pl.ds(c2 * 16, 16)]
                for k in range(16):
                    tcnt = tcnt + (bchunk[k] <= gvec).astype(jnp.int32)
            tg = tcnt - 1
            bg_ = plsc.load_gather(nbase_v, [tg])
            fg = plsc.load_gather(first_v, [tg])
            eg = fg + gvec - bg_
            eg = jnp.minimum(jnp.maximum(eg, 0), _E - 1)
            og = plsc.load_gather(offs, [eg])
            cg = plsc.load_gather(cum, [eg])
            st = jnp.maximum(og, tg * _ROW_TILE)
            en = jnp.minimum(cg, tg * _ROW_TILE + _ROW_TILE)
            valid = gvec < tot_items
            st = jnp.where(valid, st, 0)
            en = jnp.where(valid, en, 0)
            init = (gvec == bg_).astype(jnp.int32)
            t96[pl.ds(c * 16, 16)] = tg
            e96[pl.ds(c * 16, 16)] = eg
            s96[pl.ds(c * 16, 16)] = st
            en96[pl.ds(c * 16, 16)] = en
            in96[pl.ds(c * 16, 16)] = init
        pltpu.sync_copy(t96, tiles_hbm)
        pltpu.sync_copy(e96, experts_hbm)
        pltpu.sync_copy(s96, starts_hbm)
        pltpu.sync_copy(en96, ends_hbm)
        pltpu.sync_copy(in96, inits_hbm)


def _sc_dispatch(e_flat, r_flat, offs_arr, cum_arr, x):
    f32, i32 = jnp.float32, jnp.int32
    mesh = plsc.VectorSubcoreMesh(core_axis_name="c", subcore_axis_name="s")
    call = pl.kernel(
        _dispatch_body,
        out_type=(
            jax.ShapeDtypeStruct((_NP, _DIM), f32),
            jax.ShapeDtypeStruct((_NP,), i32),
            jax.ShapeDtypeStruct((_G_ITEMS,), i32),
            jax.ShapeDtypeStruct((_G_ITEMS,), i32),
            jax.ShapeDtypeStruct((_G_ITEMS,), i32),
            jax.ShapeDtypeStruct((_G_ITEMS,), i32),
            jax.ShapeDtypeStruct((_G_ITEMS,), i32),
        ),
        mesh=mesh,
        scratch_types=[
            pltpu.VMEM((_PW,), i32),
            pltpu.VMEM((_PW,), i32),
            pltpu.VMEM((_E,), i32),
            pltpu.VMEM((_E,), i32),
            pltpu.VMEM((_PW,), i32),
            pltpu.VMEM((_PW, _DIM), f32),
            pltpu.VMEM((_N_TILES,), i32),
            pltpu.VMEM((_N_TILES,), i32),
            pltpu.VMEM((16 + _N_TILES,), i32),
            pltpu.VMEM((16 + _N_TILES,), i32),
            pltpu.VMEM((_G_ITEMS,), i32),
            pltpu.VMEM((_G_ITEMS,), i32),
            pltpu.VMEM((_G_ITEMS,), i32),
            pltpu.VMEM((_G_ITEMS,), i32),
            pltpu.VMEM((_G_ITEMS,), i32),
            pltpu.SemaphoreType.DMA,
        ],
        compiler_params=pltpu.CompilerParams(needs_layout_passes=False),
    )
    return call(e_flat, r_flat, offs_arr, cum_arr, x)


def _combine_body(ys_hbm, dest_hbm, w_hbm, acc_hbm,
                  d0, d1, wv0, wv1, rows0, rows1, accv, sem0, sem1):
    wid = lax.axis_index("s") * 2 + lax.axis_index("c")
    half = _TW // 2
    for bat in range(2):
        tbase = wid * _TW + bat * half
        pltpu.sync_copy(dest_hbm.at[pl.ds(tbase, half)], d0)
        pltpu.sync_copy(dest_hbm.at[pl.ds(_T + tbase, half)], d1)
        pltpu.sync_copy(w_hbm.at[pl.ds(tbase, half)], wv0)
        pltpu.sync_copy(w_hbm.at[pl.ds(_T + tbase, half)], wv1)
        c0 = pltpu.async_copy(ys_hbm.at[d0], rows0, sem0)
        c1 = pltpu.async_copy(ys_hbm.at[d1], rows1, sem1)
        c0.wait()
        c1.wait()

        def body(t, _):
            tsplat = jnp.full((16,), t, jnp.int32)
            w0 = plsc.load_gather(wv0, [tsplat])
            w1 = plsc.load_gather(wv1, [tsplat])
            for c in range(_DIM // 16):
                sl = pl.ds(c * 16, 16)
                accv[t, sl] = rows0[t, sl] * w0 + rows1[t, sl] * w1
            return 0

        lax.fori_loop(0, half, body, 0)
        pltpu.sync_copy(accv, acc_hbm.at[pl.ds(tbase, half)])


def _sc_combine(ys, dest, w_flat):
    f32, i32 = jnp.float32, jnp.int32
    half = _TW // 2
    mesh = plsc.VectorSubcoreMesh(core_axis_name="c", subcore_axis_name="s")
    call = pl.kernel(
        _combine_body,
        out_type=jax.ShapeDtypeStruct((_T, _DIM), f32),
        mesh=mesh,
        scratch_types=[
            pltpu.VMEM((half,), i32),
            pltpu.VMEM((half,), i32),
            pltpu.VMEM((half,), f32),
            pltpu.VMEM((half,), f32),
            pltpu.VMEM((half, _DIM), f32),
            pltpu.VMEM((half, _DIM), f32),
            pltpu.VMEM((half, _DIM), f32),
            pltpu.SemaphoreType.DMA,
            pltpu.SemaphoreType.DMA,
        ],
        compiler_params=pltpu.CompilerParams(needs_layout_passes=False),
    )
    return call(ys, dest, w_flat)


def _shared_body(x_ref, ws1_ref, bs1_ref, ws3_ref, bs3_ref, ws2_ref, bs2_ref,
                 out_ref):
    out_ref[...] = _expert_ffn(x_ref[...], ws1_ref[...], bs1_ref[...],
                               ws3_ref[...], bs3_ref[...], ws2_ref[...],
                               bs2_ref[...], _C)


def _midpoint_norm(z):
    lane = lax.broadcasted_iota(jnp.int32, z.shape, 1)
    sgn = jnp.where(lane == 0, -1.0, 1.0)
    inner = jnp.sum(z * z * sgn, axis=1, keepdims=True)
    return z / jnp.sqrt(jnp.clip(-inner, 1e-8))


def _final_body(shared_ref, acc_ref, out_ref):
    routed = _midpoint_norm(acc_ref[...])
    out_ref[...] = _midpoint_norm(shared_ref[...] + routed)


def kernel(x, Wg, bg, W1, b1, W3, b3, W2, b2, Ws1, bs1, Ws3, bs3, Ws2, bs2):
    f32, i32 = jnp.float32, jnp.int32
    wg_ext = jnp.concatenate([jnp.zeros((_E, 1), f32), Wg], axis=1)
    ltri = jnp.tri(_GATE_TILE, k=-1, dtype=f32)
    eye = jnp.eye(_GATE_TILE, dtype=f32)
    l64x = jnp.tri(_E, k=-1, dtype=f32)
    l64i = jnp.tri(_E, k=0, dtype=f32)

    route, offv, cumv = pl.pallas_call(
        _gate_body,
        grid=(_N_GATE,),
        in_specs=[
            pl.BlockSpec((_GATE_TILE, _DIM), lambda i: (i, 0)),
            pl.BlockSpec((_E, _DIM), lambda i: (0, 0)),
            pl.BlockSpec((1, _E), lambda i: (0, 0)),
            pl.BlockSpec((_GATE_TILE, _GATE_TILE), lambda i: (0, 0)),
            pl.BlockSpec((_GATE_TILE, _GATE_TILE), lambda i: (0, 0)),
            pl.BlockSpec((_E, _E), lambda i: (0, 0)),
            pl.BlockSpec((_E, _E), lambda i: (0, 0)),
        ],
        out_specs=[
            pl.BlockSpec((1, 6, _GATE_TILE), lambda i: (i, 0, 0)),
            pl.BlockSpec((1, _E), lambda i: (0, 0)),
            pl.BlockSpec((1, _E), lambda i: (0, 0)),
        ],
        out_shape=[
            jax.ShapeDtypeStruct((_N_GATE, 6, _GATE_TILE), f32),
            jax.ShapeDtypeStruct((1, _E), f32),
            jax.ShapeDtypeStruct((1, _E), f32),
        ],
        scratch_shapes=[pltpu.VMEM((8, _E), f32)],
    )(x, wg_ext, bg.reshape(1, _E), ltri, eye, l64x, l64i)

    r_flat = jnp.concatenate(
        [route[:, 0, :].reshape(-1), route[:, 1, :].reshape(-1)]).astype(i32)
    e_flat = jnp.concatenate(
        [route[:, 2, :].reshape(-1), route[:, 3, :].reshape(-1)]).astype(i32)
    w_flat = jnp.concatenate(
        [route[:, 4, :].reshape(-1), route[:, 5, :].reshape(-1)])
    offs_arr = offv.reshape(_E).astype(i32)
    cum_arr = cumv.reshape(_E).astype(i32)

    xs, dest, tiles, experts, starts, ends, inits = _sc_dispatch(
        e_flat, r_flat, offs_arr, cum_arr, x)

    def _native(w):
        return w.transpose(1, 0, 2)

    w1n = _native(W1)
    w3n = _native(W3)
    w2n = _native(W2)

    fetch = jnp.concatenate(
        [jnp.ones((1,), i32), (experts[1:] != experts[:-1]).astype(i32)])
    slots = jnp.cumsum(fetch) % 2

    grid_spec = pltpu.PrefetchScalarGridSpec(
        num_scalar_prefetch=7,
        grid=(_G_ITEMS,),
        in_specs=[
            pl.BlockSpec((_ROW_TILE, _DIM),
                         lambda g, tr, er, sr, enr, ir, fr, sl: (tr[g], 0)),
            pl.BlockSpec(memory_space=pl.ANY),
            pl.BlockSpec((1, 1, _INTER - 1),
                         lambda g, tr, er, sr, enr, ir, fr, sl:
                         (er[g], 0, 0)),
            pl.BlockSpec(memory_space=pl.ANY),
            pl.BlockSpec((1, 1, _INTER - 1),
                         lambda g, tr, er, sr, enr, ir, fr, sl:
                         (er[g], 0, 0)),
            pl.BlockSpec(memory_space=pl.ANY),
            pl.BlockSpec((1, 1, _DIM - 1),
                         lambda g, tr, er, sr, enr, ir, fr, sl:
                         (er[g], 0, 0)),
        ],
        out_specs=pl.BlockSpec((_ROW_TILE, _DIM),
                               lambda g, tr, er, sr, enr, ir, fr, sl:
                               (tr[g], 0)),
        scratch_shapes=[
            pltpu.VMEM((2, _INTER - 1, _DIM), f32),
            pltpu.VMEM((2, _INTER - 1, _DIM), f32),
            pltpu.VMEM((2, _DIM - 1, _INTER), f32),
            pltpu.SemaphoreType.DMA((2,)),
            pltpu.SemaphoreType.DMA((2,)),
            pltpu.SemaphoreType.DMA((2,)),
        ],
    )
    ys = pl.pallas_call(
        _grouped_body,
        grid_spec=grid_spec,
        out_shape=jax.ShapeDtypeStruct((_NP, _DIM), f32),
    )(tiles, experts, starts, ends, inits, fetch, slots, xs,
      w1n, b1.reshape(_E, 1, _INTER - 1), w3n, b3.reshape(_E, 1, _INTER - 1),
      w2n, b2.reshape(_E, 1, _DIM - 1))

    acc = _sc_combine(ys, dest, w_flat)

    shared = pl.pallas_call(
        _shared_body,
        grid=(_T // 128,),
        in_specs=[
            pl.BlockSpec((128, _DIM), lambda i: (i, 0)),
            pl.BlockSpec((_INTER - 1, _DIM), lambda i: (0, 0)),
            pl.BlockSpec((1, _INTER - 1), lambda i: (0, 0)),
            pl.BlockSpec((_INTER - 1, _DIM), lambda i: (0, 0)),
            pl.BlockSpec((1, _INTER - 1), lambda i: (0, 0)),
            pl.BlockSpec((_DIM - 1, _INTER), lambda i: (0, 0)),
            pl.BlockSpec((1, _DIM - 1), lambda i: (0, 0)),
        ],
        out_specs=pl.BlockSpec((128, _DIM), lambda i: (i, 0)),
        out_shape=jax.ShapeDtypeStruct((_T, _DIM), f32),
    )(x, Ws1, bs1.reshape(1, _INTER - 1), Ws3, bs3.reshape(1, _INTER - 1),
      Ws2, bs2.reshape(1, _DIM - 1))

    out = pl.pallas_call(
        _final_body,
        grid=(_T // 128,),
        in_specs=[
            pl.BlockSpec((128, _DIM), lambda i: (i, 0)),
            pl.BlockSpec((128, _DIM), lambda i: (i, 0)),
        ],
        out_specs=pl.BlockSpec((128, _DIM), lambda i: (i, 0)),
        out_shape=jax.ShapeDtypeStruct((_T, _DIM), f32),
    )(shared, acc)
    return out
